# baseline (speedup 1.0000x reference)
import functools

import jax
import jax.numpy as jnp
from jax import lax
from jax.experimental import pallas as pl
from jax.experimental.pallas import tpu as pltpu
from jax.experimental.pallas import tpu_sc as plsc

BATCH = 16384
EMBED_DIM = 32
NUM_WORKERS = 32
B_PER_W = BATCH // NUM_WORKERS
CHUNK = 128
N_CHUNKS = B_PER_W // CHUNK


def _sc_body(uidx_hbm, iidx_hbm, uemb_hbm, iemb_hbm, ubias_hbm, ibias_hbm,
             out_hbm,
             uidx_v, iidx_v, urows_v, irows_v, ub_v, ib_v, work_v, work2_v,
             dot_v, out_v, sem):
    wid = lax.axis_index("s") * 2 + lax.axis_index("c")
    row0 = wid * N_CHUNKS
    base = wid * B_PER_W

    pltpu.sync_copy(uidx_hbm.at[pl.ds(row0, N_CHUNKS)], uidx_v)
    pltpu.sync_copy(iidx_hbm.at[pl.ds(row0, N_CHUNKS)], iidx_v)

    copies = []
    for j in range(N_CHUNKS):
        dst = pl.ds(j * CHUNK, CHUNK)
        copies.append(pltpu.make_async_copy(
            uemb_hbm.at[uidx_v.at[j]], urows_v.at[dst], sem))
        copies.append(pltpu.make_async_copy(
            iemb_hbm.at[iidx_v.at[j]], irows_v.at[dst], sem))
        copies.append(pltpu.make_async_copy(
            ubias_hbm.at[uidx_v.at[j]], ub_v.at[dst], sem))
        copies.append(pltpu.make_async_copy(
            ibias_hbm.at[iidx_v.at[j]], ib_v.at[dst], sem))
    for c in copies:
        c.start()
    for c in copies:
        c.wait()

    def dot_body(b, carry):
        u0 = urows_v[b, pl.ds(0, 16)]
        u1 = urows_v[b, pl.ds(16, 16)]
        v0 = irows_v[b, pl.ds(0, 16)]
        v1 = irows_v[b, pl.ds(16, 16)]
        off = pl.multiple_of(b * 16, 16)
        work_v[pl.ds(off, 16)] = u0 * v0 + u1 * v1
        return carry

    lax.fori_loop(0, B_PER_W, dot_body, 0, unroll=4)

    iota = lax.iota(jnp.int32, 16)

    def fold(src, dst, n_out):
        def body(i, carry):
            src_base = i * 32
            a = plsc.load_gather(src, [src_base + iota * 2])
            b = plsc.load_gather(src, [src_base + iota * 2 + 1])
            off = pl.multiple_of(i * 16, 16)
            dst[pl.ds(off, 16)] = a + b
            return carry

        lax.fori_loop(0, n_out // 16, body, 0, unroll=4)

    fold(work_v, work2_v, 4096)
    fold(work2_v, work_v, 2048)
    fold(work_v, work2_v, 1024)
    fold(work2_v, dot_v, 512)

    def sig_body(g, carry):
        off = pl.multiple_of(g * 16, 16)
        x = dot_v[pl.ds(off, 16)] + ub_v[pl.ds(off, 16)] + ib_v[pl.ds(off, 16)]
        out_v[pl.ds(off, 16)] = 1.0 / (1.0 + jnp.exp(-x))
        return carry

    lax.fori_loop(0, B_PER_W // 16, sig_body, 0, unroll=4)

    pltpu.sync_copy(out_v, out_hbm.at[pl.ds(base, B_PER_W)])


@jax.jit
def _mf_sc(uidx, iidx, uemb, iemb, ubias, ibias):
    mesh = plsc.VectorSubcoreMesh(core_axis_name="c", subcore_axis_name="s")
    f = functools.partial(
        pl.kernel,
        mesh=mesh,
        compiler_params=pltpu.CompilerParams(
            needs_layout_passes=False, use_tc_tiling_on_sc=False),
        out_type=jax.ShapeDtypeStruct((BATCH,), jnp.float32),
        scratch_types=[
            pltpu.VMEM((N_CHUNKS, CHUNK), jnp.int32),
            pltpu.VMEM((N_CHUNKS, CHUNK), jnp.int32),
            pltpu.VMEM((B_PER_W, EMBED_DIM), jnp.float32),
            pltpu.VMEM((B_PER_W, EMBED_DIM), jnp.float32),
            pltpu.VMEM((B_PER_W,), jnp.float32),
            pltpu.VMEM((B_PER_W,), jnp.float32),
            pltpu.VMEM((B_PER_W * 16,), jnp.float32),
            pltpu.VMEM((B_PER_W * 8,), jnp.float32),
            pltpu.VMEM((B_PER_W,), jnp.float32),
            pltpu.VMEM((B_PER_W,), jnp.float32),
            pltpu.SemaphoreType.DMA,
        ],
    )(_sc_body)
    return f(uidx, iidx, uemb, iemb, ubias, ibias)


def kernel(user_idx, item_idx, user_emb, item_emb, user_bias, item_bias):
    uidx = user_idx.astype(jnp.int32).reshape(BATCH // CHUNK, CHUNK)
    iidx = item_idx.astype(jnp.int32).reshape(BATCH // CHUNK, CHUNK)
    ubias = user_bias.reshape(-1)
    ibias = item_bias.reshape(-1)
    return _mf_sc(uidx, iidx, user_emb, item_emb, ubias, ibias)
